# baseline (speedup 1.0000x reference)
import functools

import jax
import jax.numpy as jnp
from jax import lax
from jax.experimental import pallas as pl
from jax.experimental.pallas import tpu as pltpu
from jax.experimental.pallas import tpu_sc as plsc

_INFO = plsc.get_sparse_core_info()
_NC = _INFO.num_cores
_NS = _INFO.num_subcores
_NW = _NC * _NS

_CHUNK = 128
_NBUF = 4
_LAG = 3


@functools.lru_cache(maxsize=None)
def _make_gather(V, D, Bx, T):
    B = Bx * T
    assert B % _NW == 0
    b_per_w = B // _NW
    assert b_per_w % (_CHUNK * _NBUF) == 0
    assert Bx % _CHUNK == 0 and D % 16 == 0 and T % 8 == 0
    n_chunks = b_per_w // _CHUNK
    n_groups = n_chunks // _NBUF
    chunks_per_t = Bx // _CHUNK
    mesh = plsc.VectorSubcoreMesh(core_axis_name="c", subcore_axis_name="s")

    @functools.partial(
        pl.kernel,
        mesh=mesh,
        out_type=jax.ShapeDtypeStruct((T, D // 8, Bx // _CHUNK, 8, _CHUNK),
                                      jnp.float32),
        scratch_types=[
            pltpu.VMEM((b_per_w,), jnp.int32),
            pltpu.VMEM((_NBUF, _CHUNK, D), jnp.float32),
            pltpu.VMEM((_NBUF, D, _CHUNK), jnp.float32),
            pltpu.SemaphoreType.DMA,
            pltpu.SemaphoreType.DMA((_NBUF,)),
            pltpu.SemaphoreType.DMA((_NBUF,)),
        ],
        compiler_params=pltpu.CompilerParams(
            use_tc_tiling_on_sc=False, needs_layout_passes=False
        ),
    )
    def gather_kernel(idx_hbm, table_hbm, out_hbm, idx_v, rows_v, tbuf_v,
                      isem, gsem, ssem):
        wid = lax.axis_index("s") * _NC + lax.axis_index("c")
        base = wid * b_per_w
        chunk0 = wid * n_chunks
        pltpu.async_copy(idx_hbm.at[pl.ds(base, b_per_w)], idx_v, isem).wait()

        def start_gather(chunk, b):
            pltpu.async_copy(
                table_hbm.at[idx_v.at[pl.ds(chunk * _CHUNK, _CHUNK)]],
                rows_v.at[b],
                gsem.at[b],
            )

        def wait_gather(b):
            pltpu.make_async_copy(
                table_hbm.at[idx_v.at[pl.ds(0, _CHUNK)]], rows_v.at[b],
                gsem.at[b],
            ).wait()

        def start_store(chunk, b):
            g = chunk0 + chunk
            t8 = g // (chunks_per_t * 8)
            b1 = (g // 8) % chunks_per_t
            t1 = g % 8
            t = t8 * 8 + t1
            bo = b1
            for co in range(D // 8):
                pltpu.async_copy(
                    tbuf_v.at[b, pl.ds(co * 8, 8)],
                    out_hbm.at[t, co, bo],
                    ssem.at[b],
                )

        def wait_store(b):
            for co in range(D // 8):
                pltpu.make_async_copy(
                    tbuf_v.at[b, pl.ds(0, 8)], out_hbm.at[0, 0, 0], ssem.at[b]
                ).wait()

        def transpose(b):
            rows = rows_v.at[b]
            tb = tbuf_v.at[b]
            lane = lax.iota(jnp.int32, 16)

            def r_body(r0, carry):
                ridx = lane + r0 * 16
                for c0 in range(0, D, 16):
                    for j in range(16):
                        cidx = ((lane + j) & 15) + c0
                        vals = plsc.load_gather(rows, [ridx, cidx])
                        plsc.store_scatter(tb, [cidx, ridx], vals)
                return carry

            lax.fori_loop(0, _CHUNK // 16, r_body, 0)

        for b in range(_LAG):
            start_gather(b, b)

        for b in range(_NBUF):
            wait_gather(b)
            transpose(b)
            start_store(b, b)
            start_gather(b + _LAG, (b + _LAG) % _NBUF)

        def group_body(g, carry):
            for b in range(_NBUF):
                i = g * _NBUF + b
                wait_gather(b)
                wait_store(b)
                transpose(b)
                start_store(i, b)
                j = i + _LAG
                bj = (b + _LAG) % _NBUF

                @pl.when(j < n_chunks)
                def _():
                    start_gather(j, bj)

            return carry

        lax.fori_loop(1, n_groups, group_body, 0)

        for b in range(_NBUF):
            wait_store(b)

    return gather_kernel


def kernel(x, weight):
    Bx, T = x.shape
    V, D = weight.shape
    x4 = x.reshape(Bx // _CHUNK, _CHUNK, T // 8, 8)
    idx = x4.transpose((2, 0, 3, 1)).reshape(Bx * T).astype(jnp.int32)
    out5 = _make_gather(V, D, Bx, T)(idx, weight)
    return out5.transpose((2, 4, 0, 1, 3)).reshape(Bx, T, D)


# speedup vs baseline: 1.4427x; 1.4427x over previous
import functools

import jax
import jax.numpy as jnp
from jax import lax
from jax.experimental import pallas as pl
from jax.experimental.pallas import tpu as pltpu
from jax.experimental.pallas import tpu_sc as plsc

_INFO = plsc.get_sparse_core_info()
_NC = _INFO.num_cores
_NS = _INFO.num_subcores
_NW = _NC * _NS

_CHUNK = 128
_NBUF = 4
_LAG = 3


@functools.lru_cache(maxsize=None)
def _make_gather(V, D, Bx, T):
    B = Bx * T
    assert B % _NW == 0
    b_per_w = B // _NW
    assert b_per_w % (_CHUNK * _NBUF) == 0
    assert Bx % _CHUNK == 0 and D % 16 == 0 and T % 8 == 0
    n_chunks = b_per_w // _CHUNK
    n_groups = n_chunks // _NBUF
    chunks_per_t = Bx // _CHUNK
    mesh = plsc.VectorSubcoreMesh(core_axis_name="c", subcore_axis_name="s")

    @functools.partial(
        pl.kernel,
        mesh=mesh,
        out_type=jax.ShapeDtypeStruct((T, D // 8, Bx // _CHUNK, 8, _CHUNK),
                                      jnp.float32),
        scratch_types=[
            pltpu.VMEM((b_per_w,), jnp.int32),
            pltpu.VMEM((_NBUF, _CHUNK, D), jnp.float32),
            pltpu.VMEM((_NBUF, D, _CHUNK), jnp.float32),
            pltpu.SemaphoreType.DMA,
            pltpu.SemaphoreType.DMA((_NBUF,)),
            pltpu.SemaphoreType.DMA((_NBUF,)),
        ],
        compiler_params=pltpu.CompilerParams(
            use_tc_tiling_on_sc=False, needs_layout_passes=False
        ),
    )
    def gather_kernel(idx_hbm, table_hbm, out_hbm, idx_v, rows_v, tbuf_v,
                      isem, gsem, ssem):
        wid = lax.axis_index("s") * _NC + lax.axis_index("c")
        base = wid * b_per_w
        chunk0 = wid * n_chunks
        pltpu.async_copy(idx_hbm.at[pl.ds(base, b_per_w)], idx_v, isem).wait()

        def start_gather(chunk, b):
            pltpu.async_copy(
                table_hbm.at[idx_v.at[pl.ds(chunk * _CHUNK, _CHUNK)]],
                rows_v.at[b],
                gsem.at[b],
            )

        def wait_gather(b):
            pltpu.make_async_copy(
                table_hbm.at[idx_v.at[pl.ds(0, _CHUNK)]], rows_v.at[b],
                gsem.at[b],
            ).wait()

        def start_store(chunk, b):
            g = chunk0 + chunk
            t8 = g // (chunks_per_t * 8)
            b1 = (g // 8) % chunks_per_t
            t1 = g % 8
            t = t8 * 8 + t1
            bo = b1
            for co in range(D // 8):
                pltpu.async_copy(
                    tbuf_v.at[b, pl.ds(co * 8, 8)],
                    out_hbm.at[t, co, bo],
                    ssem.at[b],
                )

        def wait_store(b):
            for co in range(D // 8):
                pltpu.make_async_copy(
                    tbuf_v.at[b, pl.ds(0, 8)], out_hbm.at[0, 0, 0], ssem.at[b]
                ).wait()

        def transpose(b):
            rows = rows_v.at[b]
            tb = tbuf_v.at[b]
            lane = lax.iota(jnp.int32, 16)

            def r_body(r0, carry):
                ridx = lane + r0 * 16
                for c0 in range(0, D, 16):
                    cidxs = [((lane + j) & 15) + c0 for j in range(16)]
                    vals = [plsc.load_gather(rows, [ridx, cidxs[j]])
                            for j in range(16)]
                    for j in range(16):
                        plsc.store_scatter(tb, [cidxs[j], ridx], vals[j])
                return carry

            lax.fori_loop(0, _CHUNK // 16, r_body, 0)

        for b in range(_LAG):
            start_gather(b, b)

        for b in range(_NBUF):
            wait_gather(b)
            transpose(b)
            start_store(b, b)
            start_gather(b + _LAG, (b + _LAG) % _NBUF)

        def group_body(g, carry):
            for b in range(_NBUF):
                i = g * _NBUF + b
                wait_gather(b)
                wait_store(b)
                transpose(b)
                start_store(i, b)
                j = i + _LAG
                bj = (b + _LAG) % _NBUF

                @pl.when(j < n_chunks)
                def _():
                    start_gather(j, bj)

            return carry

        lax.fori_loop(1, n_groups, group_body, 0)

        for b in range(_NBUF):
            wait_store(b)

    return gather_kernel


def kernel(x, weight):
    Bx, T = x.shape
    V, D = weight.shape
    x4 = x.reshape(Bx // _CHUNK, _CHUNK, T // 8, 8)
    idx = x4.transpose((2, 0, 3, 1)).reshape(Bx * T).astype(jnp.int32)
    out5 = _make_gather(V, D, Bx, T)(idx, weight)
    return out5.transpose((2, 4, 0, 1, 3)).reshape(Bx, T, D)


# speedup vs baseline: 1.5292x; 1.0599x over previous
import functools

import jax
import jax.numpy as jnp
from jax import lax
from jax.experimental import pallas as pl
from jax.experimental.pallas import tpu as pltpu
from jax.experimental.pallas import tpu_sc as plsc

_INFO = plsc.get_sparse_core_info()
_NC = _INFO.num_cores
_NS = _INFO.num_subcores
_NW = _NC * _NS

_CHUNK = 128
_NBUF = 4
_LAG = 3


@functools.lru_cache(maxsize=None)
def _make_gather(V, D, Bx, T, W):
    B = Bx * T
    assert B % _NW == 0
    b_per_w = B // _NW
    assert b_per_w % (_CHUNK * _NBUF) == 0
    assert Bx % _CHUNK == 0 and D % 16 == 0 and T % 8 == 0
    n_chunks = b_per_w // _CHUNK
    n_groups = n_chunks // _NBUF
    chunks_per_t = Bx // _CHUNK
    mesh = plsc.VectorSubcoreMesh(core_axis_name="c", subcore_axis_name="s")

    @functools.partial(
        pl.kernel,
        mesh=mesh,
        out_type=jax.ShapeDtypeStruct((T, D // 8, Bx // _CHUNK, 8, _CHUNK),
                                      jnp.float32),
        scratch_types=[
            pltpu.VMEM((b_per_w,), jnp.int32),
            pltpu.VMEM((_NBUF, _CHUNK, W), jnp.float32),
            pltpu.VMEM((_NBUF, D, _CHUNK), jnp.float32),
            pltpu.SemaphoreType.DMA,
            pltpu.SemaphoreType.DMA((_NBUF,)),
            pltpu.SemaphoreType.DMA((_NBUF,)),
        ],
        compiler_params=pltpu.CompilerParams(
            use_tc_tiling_on_sc=False, needs_layout_passes=False
        ),
    )
    def gather_kernel(idx_hbm, table_hbm, out_hbm, idx_v, rows_v, tbuf_v,
                      isem, gsem, ssem):
        wid = lax.axis_index("s") * _NC + lax.axis_index("c")
        base = wid * b_per_w
        chunk0 = wid * n_chunks
        pltpu.async_copy(idx_hbm.at[pl.ds(base, b_per_w)], idx_v, isem).wait()

        def start_gather(chunk, b):
            pltpu.async_copy(
                table_hbm.at[idx_v.at[pl.ds(chunk * _CHUNK, _CHUNK)]],
                rows_v.at[b],
                gsem.at[b],
            )

        def wait_gather(b):
            pltpu.make_async_copy(
                table_hbm.at[idx_v.at[pl.ds(0, _CHUNK)]], rows_v.at[b],
                gsem.at[b],
            ).wait()

        def start_store(chunk, b):
            g = chunk0 + chunk
            t8 = g // (chunks_per_t * 8)
            b1 = (g // 8) % chunks_per_t
            t1 = g % 8
            t = t8 * 8 + t1
            bo = b1
            for co in range(D // 8):
                pltpu.async_copy(
                    tbuf_v.at[b, pl.ds(co * 8, 8)],
                    out_hbm.at[t, co, bo],
                    ssem.at[b],
                )

        def wait_store(b):
            for co in range(D // 8):
                pltpu.make_async_copy(
                    tbuf_v.at[b, pl.ds(0, 8)], out_hbm.at[0, 0, 0], ssem.at[b]
                ).wait()

        def transpose(b):
            rows = rows_v.at[b]
            tb = tbuf_v.at[b]
            lane = lax.iota(jnp.int32, 16)

            def r_body(r0, carry):
                ridx = lane + r0 * 16
                for c0 in range(0, D, 16):
                    cidxs = [((lane + j) & 15) + c0 for j in range(16)]
                    vals = [plsc.load_gather(rows, [ridx, cidxs[j]])
                            for j in range(16)]
                    for j in range(16):
                        plsc.store_scatter(tb, [cidxs[j], ridx], vals[j])
                return carry

            lax.fori_loop(0, _CHUNK // 16, r_body, 0)

        for b in range(_LAG):
            start_gather(b, b)

        for b in range(_NBUF):
            wait_gather(b)
            transpose(b)
            start_store(b, b)
            start_gather(b + _LAG, (b + _LAG) % _NBUF)

        def group_body(g, carry):
            for b in range(_NBUF):
                i = g * _NBUF + b
                wait_gather(b)
                wait_store(b)
                transpose(b)
                start_store(i, b)
                j = i + _LAG
                bj = (b + _LAG) % _NBUF

                @pl.when(j < n_chunks)
                def _():
                    start_gather(j, bj)

            return carry

        lax.fori_loop(1, n_groups, group_body, 0)

        for b in range(_NBUF):
            wait_store(b)

    return gather_kernel


def kernel(x, weight):
    Bx, T = x.shape
    V, D = weight.shape
    x4 = x.reshape(Bx // _CHUNK, _CHUNK, T // 8, 8)
    idx = x4.transpose((2, 0, 3, 1)).reshape(Bx * T).astype(jnp.int32)
    W = 128
    wp = jnp.pad(weight, ((0, 0), (0, W - D)))
    out5 = _make_gather(V, D, Bx, T, W)(idx, wp)
    return out5.transpose((2, 4, 0, 1, 3)).reshape(Bx, T, D)


# speedup vs baseline: 1.5412x; 1.0079x over previous
import functools

import jax
import jax.numpy as jnp
from jax import lax
from jax.experimental import pallas as pl
from jax.experimental.pallas import tpu as pltpu
from jax.experimental.pallas import tpu_sc as plsc

_INFO = plsc.get_sparse_core_info()
_NC = _INFO.num_cores
_NS = _INFO.num_subcores
_NW = _NC * _NS

_CHUNK = 128
_NBUF = 4
_LAG = 3


@functools.lru_cache(maxsize=None)
def _make_gather(V, D, Bx, T, W):
    B = Bx * T
    assert B % _NW == 0
    b_per_w = B // _NW
    assert b_per_w % (_CHUNK * _NBUF) == 0
    assert Bx % _CHUNK == 0 and D % 16 == 0 and T % 8 == 0
    n_chunks = b_per_w // _CHUNK
    n_groups = n_chunks // _NBUF
    chunks_per_t = Bx // _CHUNK
    mesh = plsc.VectorSubcoreMesh(core_axis_name="c", subcore_axis_name="s")

    @functools.partial(
        pl.kernel,
        mesh=mesh,
        out_type=jax.ShapeDtypeStruct((T, D // 8, Bx // _CHUNK, 8, _CHUNK),
                                      jnp.float32),
        scratch_types=[
            pltpu.VMEM((b_per_w,), jnp.int32),
            pltpu.VMEM((b_per_w,), jnp.int32),
            pltpu.VMEM((_NBUF, _CHUNK, D), jnp.float32),
            pltpu.VMEM((_NBUF, D, _CHUNK), jnp.float32),
            pltpu.SemaphoreType.DMA,
            pltpu.SemaphoreType.DMA((_NBUF,)),
            pltpu.SemaphoreType.DMA((_NBUF,)),
        ],
        compiler_params=pltpu.CompilerParams(
            use_tc_tiling_on_sc=False, needs_layout_passes=False
        ),
    )
    def gather_kernel(idx_hbm, table_hbm, out_hbm, idx_v, idx2_v, rows_v,
                      tbuf_v, isem, gsem, ssem):
        wid = lax.axis_index("s") * _NC + lax.axis_index("c")
        base = wid * b_per_w
        chunk0 = wid * n_chunks
        pltpu.async_copy(idx_hbm.at[pl.ds(base, b_per_w)], idx_v, isem).wait()

        def dbl_body(k, carry):
            v = idx_v[pl.ds(k * 16, 16)]
            idx2_v[pl.ds(k * 16, 16)] = v + v
            return carry

        lax.fori_loop(0, b_per_w // 16, dbl_body, 0)

        def start_gather(chunk, b):
            pltpu.async_copy(
                table_hbm.at[idx2_v.at[pl.ds(chunk * _CHUNK, _CHUNK)]],
                rows_v.at[b],
                gsem.at[b],
            )

        def wait_gather(b):
            pltpu.make_async_copy(
                table_hbm.at[idx2_v.at[pl.ds(0, _CHUNK)]], rows_v.at[b],
                gsem.at[b],
            ).wait()

        def start_store(chunk, b):
            g = chunk0 + chunk
            t8 = g // (chunks_per_t * 8)
            b1 = (g // 8) % chunks_per_t
            t1 = g % 8
            t = t8 * 8 + t1
            bo = b1
            for co in range(D // 8):
                pltpu.async_copy(
                    tbuf_v.at[b, pl.ds(co * 8, 8)],
                    out_hbm.at[t, co, bo],
                    ssem.at[b],
                )

        def wait_store(b):
            for co in range(D // 8):
                pltpu.make_async_copy(
                    tbuf_v.at[b, pl.ds(0, 8)], out_hbm.at[0, 0, 0], ssem.at[b]
                ).wait()

        def transpose(b):
            rows = rows_v.at[b]
            tb = tbuf_v.at[b]
            lane = lax.iota(jnp.int32, 16)

            def r_body(r0, carry):
                ridx = lane + r0 * 16
                for c0 in range(0, D, 16):
                    cidxs = [((lane + j) & 15) + c0 for j in range(16)]
                    vals = [plsc.load_gather(rows, [ridx, cidxs[j]])
                            for j in range(16)]
                    for j in range(16):
                        plsc.store_scatter(tb, [cidxs[j], ridx], vals[j])
                return carry

            lax.fori_loop(0, _CHUNK // 16, r_body, 0)

        for b in range(_LAG):
            start_gather(b, b)

        for b in range(_NBUF):
            wait_gather(b)
            transpose(b)
            start_store(b, b)
            start_gather(b + _LAG, (b + _LAG) % _NBUF)

        def group_body(g, carry):
            for b in range(_NBUF):
                i = g * _NBUF + b
                wait_gather(b)
                wait_store(b)
                transpose(b)
                start_store(i, b)
                j = i + _LAG
                bj = (b + _LAG) % _NBUF

                @pl.when(j < n_chunks)
                def _():
                    start_gather(j, bj)

            return carry

        lax.fori_loop(1, n_groups, group_body, 0)

        for b in range(_NBUF):
            wait_store(b)

    return gather_kernel


def kernel(x, weight):
    Bx, T = x.shape
    V, D = weight.shape
    x4 = x.reshape(Bx // _CHUNK, _CHUNK, T // 8, 8)
    idx = x4.transpose((2, 0, 3, 1)).reshape(Bx * T).astype(jnp.int32)
    W = 128
    wp = jnp.pad(weight, ((0, 0), (0, W - D))).reshape(V * (W // D), D)
    out5 = _make_gather(V, D, Bx, T, W)(idx, wp)
    return out5.transpose((2, 4, 0, 1, 3)).reshape(Bx, T, D)


# speedup vs baseline: 2.4770x; 1.6072x over previous
import functools

import jax
import jax.numpy as jnp
from jax import lax
from jax.experimental import pallas as pl
from jax.experimental.pallas import tpu as pltpu
from jax.experimental.pallas import tpu_sc as plsc

_INFO = plsc.get_sparse_core_info()
_NC = _INFO.num_cores
_NS = _INFO.num_subcores
_NW = _NC * _NS

_CHUNK = 128
_NBUF = 4
_LAG = 3


@functools.lru_cache(maxsize=None)
def _make_gather(V, D, Bx, T, W):
    B = Bx * T
    assert B % _NW == 0
    b_per_w = B // _NW
    assert b_per_w % (_CHUNK * _NBUF) == 0
    assert Bx % _CHUNK == 0 and D % 16 == 0 and T % 8 == 0
    n_chunks = b_per_w // _CHUNK
    n_groups = n_chunks // _NBUF
    chunks_per_t = Bx // _CHUNK
    mesh = plsc.VectorSubcoreMesh(core_axis_name="c", subcore_axis_name="s")

    @functools.partial(
        pl.kernel,
        mesh=mesh,
        out_type=jax.ShapeDtypeStruct((T, D // 8, Bx // _CHUNK, 8, _CHUNK),
                                      jnp.float32),
        scratch_types=[
            pltpu.VMEM((b_per_w,), jnp.int32),
            pltpu.VMEM((b_per_w,), jnp.int32),
            pltpu.VMEM((_NBUF, _CHUNK, D), jnp.float32),
            pltpu.VMEM((_NBUF, D, _CHUNK), jnp.float32),
            pltpu.SemaphoreType.DMA,
            pltpu.SemaphoreType.DMA((_NBUF,)),
            pltpu.SemaphoreType.DMA((_NBUF,)),
        ],
        compiler_params=pltpu.CompilerParams(
            use_tc_tiling_on_sc=False, needs_layout_passes=False
        ),
    )
    def gather_kernel(idx_hbm, table_hbm, out_hbm, idx_v, idx2_v, rows_v,
                      tbuf_v, isem, gsem, ssem):
        wid = lax.axis_index("s") * _NC + lax.axis_index("c")
        base = wid * b_per_w
        chunk0 = wid * n_chunks
        pltpu.async_copy(idx_hbm.at[pl.ds(base, b_per_w)], idx_v, isem).wait()

        def dbl_body(k, carry):
            v = idx_v[pl.ds(k * 16, 16)]
            idx2_v[pl.ds(k * 16, 16)] = v + v
            return carry

        lax.fori_loop(0, b_per_w // 16, dbl_body, 0)

        def start_gather(chunk, b):
            pltpu.async_copy(
                table_hbm.at[idx2_v.at[pl.ds(chunk * _CHUNK, _CHUNK)]],
                rows_v.at[b],
                gsem.at[b],
            )

        def wait_gather(b):
            pltpu.make_async_copy(
                table_hbm.at[idx2_v.at[pl.ds(0, _CHUNK)]], rows_v.at[b],
                gsem.at[b],
            ).wait()

        def start_store(chunk, b):
            g = chunk0 + chunk
            t8 = g // (chunks_per_t * 8)
            b1 = (g // 8) % chunks_per_t
            t1 = g % 8
            t = t8 * 8 + t1
            bo = b1
            for co in range(D // 8):
                pltpu.async_copy(
                    tbuf_v.at[b, pl.ds(co * 8, 8)],
                    out_hbm.at[t, co, bo],
                    ssem.at[b],
                )

        def wait_store(b):
            for co in range(D // 8):
                pltpu.make_async_copy(
                    tbuf_v.at[b, pl.ds(0, 8)], out_hbm.at[0, 0, 0], ssem.at[b]
                ).wait()

        def transpose(b):
            rows = rows_v.at[b]
            tb = tbuf_v.at[b]
            lane = lax.iota(jnp.int32, 16)

            def r_body(r0, carry):
                ridx = lane + r0 * 16
                for c0 in range(0, D, 16):
                    cidxs = [((lane + j) & 15) + c0 for j in range(16)]
                    vals = [plsc.load_gather(rows, [ridx, cidxs[j]])
                            for j in range(16)]
                    for j in range(16):
                        plsc.store_scatter(tb, [cidxs[j], ridx], vals[j])
                return carry

            lax.fori_loop(0, _CHUNK // 16, r_body, 0)

        for b in range(_LAG):
            start_gather(b, b)

        for b in range(_NBUF):
            wait_gather(b)
            transpose(b)
            start_store(b, b)
            start_gather(b + _LAG, (b + _LAG) % _NBUF)

        def group_body(g, carry):
            for b in range(_NBUF):
                i = g * _NBUF + b
                wait_gather(b)
                wait_store(b)
                transpose(b)
                start_store(i, b)
                j = i + _LAG
                bj = (b + _LAG) % _NBUF

                @pl.when(j < n_chunks)
                def _():
                    start_gather(j, bj)

            return carry

        lax.fori_loop(1, n_groups, group_body, 0)

        for b in range(_NBUF):
            wait_store(b)

    return gather_kernel


def kernel(x, weight):
    Bx, T = x.shape
    V, D = weight.shape
    x4 = x.reshape(Bx // _CHUNK, _CHUNK, T // 8, 8)
    idx = x4.transpose((2, 0, 3, 1)).reshape(Bx * T).astype(jnp.int32)
    W = 128
    proj = jnp.concatenate(
        [jnp.eye(D, dtype=weight.dtype),
         jnp.zeros((D, W - D), dtype=weight.dtype)], axis=1)
    wp = (weight @ proj).reshape(V * (W // D), D)
    out5 = _make_gather(V, D, Bx, T, W)(idx, wp)
    return out5.transpose((2, 4, 0, 1, 3)).reshape(Bx, T, D)
